# baseline (speedup 1.0000x reference)
import functools

import jax
import jax.numpy as jnp
import numpy as np
from jax import lax
from jax.experimental import pallas as pl
from jax.experimental.pallas import tpu as pltpu
from jax.experimental.pallas import tpu_sc as plsc

N_SRC = 10000
N_DST = 10000
E = 320000
D = 128
D_EDGE = 16
HEADS = 8
D_HEAD = 16
TW = 2 * D
TWW = TW // 2

NC, NS = 2, 16
NW = NC * NS
EH = E
EPW = EH // NW
EPS = EH // NS

C1 = 80
T1 = EPW % C1
C2 = 80
ZC = 80
NZCH = N_DST // ZC


def _pack16(a, b):
    au = lax.bitcast_convert_type(a.astype(jnp.bfloat16), jnp.uint16)
    bu = lax.bitcast_convert_type(b.astype(jnp.bfloat16), jnp.uint16)
    word = au.astype(jnp.uint32) | (bu.astype(jnp.uint32) << 16)
    return lax.bitcast_convert_type(word, jnp.float32)


def _unpack16(w):
    word = lax.bitcast_convert_type(w, jnp.uint32)
    a = lax.bitcast_convert_type((word & 0xFFFF).astype(jnp.uint16),
                                 jnp.bfloat16)
    b = lax.bitcast_convert_type((word >> 16).astype(jnp.uint16),
                                 jnp.bfloat16)
    return a.astype(jnp.float32), b.astype(jnp.float32)


def _tables_body(xs_ref, xd_ref, wsk_ref, wsv_ref, wdk_ref, wdv_ref,
                 bk_ref, bv_ref, ts_ref, td_ref):
    xs = xs_ref[...]
    xd = xd_ref[...]
    ts_ref[...] = _pack16(
        jnp.dot(xs, wsk_ref[...], preferred_element_type=jnp.float32),
        jnp.dot(xs, wsv_ref[...], preferred_element_type=jnp.float32))
    td_ref[...] = _pack16(
        jnp.dot(xd, wdk_ref[...], preferred_element_type=jnp.float32)
        + bk_ref[...],
        jnp.dot(xd, wdv_ref[...], preferred_element_type=jnp.float32)
        + bv_ref[...])


def _tc_tables(x_src, x_dst, wsk, wsv, wdk, wdv, bk, bv):
    blk = 1000
    grid = (N_SRC // blk,)
    return pl.pallas_call(
        _tables_body,
        grid=grid,
        in_specs=[
            pl.BlockSpec((blk, D), lambda i: (i, 0)),
            pl.BlockSpec((blk, D), lambda i: (i, 0)),
            pl.BlockSpec((D, D), lambda i: (0, 0)),
            pl.BlockSpec((D, D), lambda i: (0, 0)),
            pl.BlockSpec((D, D), lambda i: (0, 0)),
            pl.BlockSpec((D, D), lambda i: (0, 0)),
            pl.BlockSpec((1, D), lambda i: (0, 0)),
            pl.BlockSpec((1, D), lambda i: (0, 0)),
        ],
        out_specs=[
            pl.BlockSpec((blk, TWW), lambda i: (i, 0)),
            pl.BlockSpec((blk, TWW), lambda i: (i, 0)),
        ],
        out_shape=[
            jax.ShapeDtypeStruct((N_SRC, TWW), jnp.float32),
            jax.ShapeDtypeStruct((N_DST, TWW), jnp.float32),
        ],
    )(x_src, x_dst, wsk, wsv, wdk, wdv, bk, bv)


def _sc_gather_body(ts_hbm, td_hbm, src_hbm, dst_hbm, gs_hbm, gd_hbm,
                    isa, ida, bs0, bd0, bs1, bd1,
                    semg0, semg1, semw0, semw1):
    wid = lax.axis_index("s") * NC + lax.axis_index("c")
    nch = EPW // C1
    base_w = wid * EPW
    pltpu.sync_copy(src_hbm.at[pl.ds(base_w, EPW)], isa)
    pltpu.sync_copy(dst_hbm.at[pl.ds(base_w, EPW)], ida)

    def start(i, bs, bd, semg):
        pltpu.async_copy(ts_hbm.at[isa.at[pl.ds(i * C1, C1)]], bs, semg)
        pltpu.async_copy(td_hbm.at[ida.at[pl.ds(i * C1, C1)]], bd, semg)

    def finish(i, bs, bd, semg, semw):
        pltpu.make_async_copy(
            ts_hbm.at[isa.at[pl.ds(i * C1, C1)]], bs, semg).wait()
        pltpu.make_async_copy(
            td_hbm.at[ida.at[pl.ds(i * C1, C1)]], bd, semg).wait()
        sl = pl.ds(base_w + i * C1, C1)
        pltpu.async_copy(bs, gs_hbm.at[sl], semw)
        pltpu.async_copy(bd, gd_hbm.at[sl], semw)

    def drain(i, bs, bd, semw):
        sl = pl.ds(base_w + i * C1, C1)
        pltpu.make_async_copy(bs, gs_hbm.at[sl], semw).wait()
        pltpu.make_async_copy(bd, gd_hbm.at[sl], semw).wait()

    def work(i, bsA, bdA, semgA, semwA, bsB, bdB, semgB, semwB):
        @pl.when(i >= 2)
        def _():
            drain(i - 2, bsA, bdA, semwA)

        start(i, bsA, bdA, semgA)

        @pl.when(i >= 1)
        def _():
            finish(i - 1, bsB, bdB, semgB, semwB)

    def body(i, _):
        @pl.when(i % 2 == 0)
        def _():
            work(i, bs0, bd0, semg0, semw0, bs1, bd1, semg1, semw1)

        @pl.when(i % 2 == 1)
        def _():
            work(i, bs1, bd1, semg1, semw1, bs0, bd0, semg0, semw0)

        return 0

    lax.fori_loop(0, nch, body, 0)
    if (nch - 1) % 2 == 0:
        finish(nch - 1, bs0, bd0, semg0, semw0)
        drain(nch - 2, bs1, bd1, semw1)
        drain(nch - 1, bs0, bd0, semw0)
    else:
        finish(nch - 1, bs1, bd1, semg1, semw1)
        drain(nch - 2, bs0, bd0, semw0)
        drain(nch - 1, bs1, bd1, semw1)
    if T1:
        t0 = nch * C1
        pltpu.async_copy(ts_hbm.at[isa.at[pl.ds(t0, T1)]],
                         bs0.at[pl.ds(0, T1)], semg0).wait()
        pltpu.async_copy(td_hbm.at[ida.at[pl.ds(t0, T1)]],
                         bd0.at[pl.ds(0, T1)], semg0).wait()
        pltpu.sync_copy(bs0.at[pl.ds(0, T1)], gs_hbm.at[pl.ds(base_w + t0, T1)])
        pltpu.sync_copy(bd0.at[pl.ds(0, T1)], gd_hbm.at[pl.ds(base_w + t0, T1)])


def _sc_gather(ts, td, src, dst):
    mesh = plsc.VectorSubcoreMesh(core_axis_name="c", subcore_axis_name="s")
    f = functools.partial(
        pl.kernel,
        mesh=mesh,
        out_type=[
            jax.ShapeDtypeStruct((EH, TWW), jnp.float32),
            jax.ShapeDtypeStruct((EH, TWW), jnp.float32),
        ],
        scratch_types=[
            pltpu.VMEM((EPW,), jnp.int32),
            pltpu.VMEM((EPW,), jnp.int32),
            pltpu.VMEM((C1, TWW), jnp.float32),
            pltpu.VMEM((C1, TWW), jnp.float32),
            pltpu.VMEM((C1, TWW), jnp.float32),
            pltpu.VMEM((C1, TWW), jnp.float32),
            pltpu.SemaphoreType.DMA,
            pltpu.SemaphoreType.DMA,
            pltpu.SemaphoreType.DMA,
            pltpu.SemaphoreType.DMA,
        ],
    )(_sc_gather_body)
    return f(ts, td, src, dst)


def _edge_body(gs_ref, gd_ref, ea_ref, wke_ref, wve_ref, qk_ref, ck_ref,
               wv1_ref, bv1_ref, rep_ref, mn_ref, ex_ref):
    ea = ea_ref[...]
    ks, vs = _unpack16(gs_ref[...])
    kd, vd = _unpack16(gd_ref[...])
    h0k = jnp.maximum(
        ks + kd
        + jnp.dot(ea, wke_ref[...], preferred_element_type=jnp.float32), 0.0)
    coef = jnp.dot(h0k, qk_ref[...],
                   preferred_element_type=jnp.float32) + ck_ref[...]
    ex = jnp.exp(coef)
    h0v = jnp.maximum(
        vs + vd
        + jnp.dot(ea, wve_ref[...], preferred_element_type=jnp.float32), 0.0)
    v1 = jnp.dot(h0v, wv1_ref[...],
                 preferred_element_type=jnp.float32) + bv1_ref[...] + h0v
    exw = jnp.dot(ex, rep_ref[...],
                  preferred_element_type=jnp.float32)
    mn_ref[...] = exw * v1
    ex_ref[...] = ex


def _tc_edges(gs, gd, ea, wke, wve, qk, ck, wv1, bv1, rep):
    blk = 1000
    grid = (EH // blk,)
    return pl.pallas_call(
        _edge_body,
        grid=grid,
        in_specs=[
            pl.BlockSpec((blk, TWW), lambda i: (i, 0)),
            pl.BlockSpec((blk, TWW), lambda i: (i, 0)),
            pl.BlockSpec((blk, D_EDGE), lambda i: (i, 0)),
            pl.BlockSpec((D_EDGE, D), lambda i: (0, 0)),
            pl.BlockSpec((D_EDGE, D), lambda i: (0, 0)),
            pl.BlockSpec((D, HEADS), lambda i: (0, 0)),
            pl.BlockSpec((1, HEADS), lambda i: (0, 0)),
            pl.BlockSpec((D, D), lambda i: (0, 0)),
            pl.BlockSpec((1, D), lambda i: (0, 0)),
            pl.BlockSpec((HEADS, D), lambda i: (0, 0)),
        ],
        out_specs=[
            pl.BlockSpec((blk, D), lambda i: (i, 0)),
            pl.BlockSpec((blk, HEADS), lambda i: (i, 0)),
        ],
        out_shape=[
            jax.ShapeDtypeStruct((EH, D), jnp.float32),
            jax.ShapeDtypeStruct((EH, HEADS), jnp.float32),
        ],
    )(gs, gd, ea, wke, wve, qk, ck, wv1, bv1, rep)


def _sc_scatter_body(mn_hbm, ex_hbm, dst2_hbm, z_hbm, out_hbm,
                     idx2, rows0, exb0, rows1, exb1,
                     semm0, semm1, sema0, sema1, acc):
    c = lax.axis_index("c")
    s = lax.axis_index("s")
    nch = EPS // 2 // C2

    pltpu.sync_copy(z_hbm, rows0)
    pltpu.sync_copy(z_hbm, rows1)

    def zinit(j, _):
        @pl.when(j % NS == s)
        def _():
            pltpu.sync_copy(rows0, acc.at[pl.ds(j * ZC, ZC)])
        return 0

    lax.fori_loop(0, NZCH, zinit, 0)
    plsc.subcore_barrier()

    low = lax.iota(jnp.int32, 16) < HEADS

    def run_page(p):
        pltpu.sync_copy(dst2_hbm.at[s * 2 + p], idx2)
        pbase = s * EPS + p * (EPS // 2)

        def start(i, rows, exb, semm):
            base = pbase + i * C2

            @pl.when(c == 0)
            def _():
                pltpu.async_copy(mn_hbm.at[pl.ds(base, C2)], rows, semm)

            @pl.when(c == 1)
            def _():
                pltpu.async_copy(ex_hbm.at[pl.ds(base * HEADS, C2 * HEADS)],
                                 exb.at[pl.ds(0, C2 * HEADS)], semm)

        def finish(i, rows, exb, semm, sema):
            base = pbase + i * C2

            @pl.when(c == 0)
            def _():
                pltpu.make_async_copy(
                    mn_hbm.at[pl.ds(base, C2)], rows, semm).wait()

            @pl.when(c == 1)
            def _():
                pltpu.make_async_copy(
                    ex_hbm.at[pl.ds(base * HEADS, C2 * HEADS)],
                    exb.at[pl.ds(0, C2 * HEADS)], semm).wait()

                def expand(r, _):
                    vec = jnp.where(low, exb[pl.ds(r * HEADS, 16)], 0.0)
                    rows[r, pl.ds(0, 16)] = vec
                    return 0

                lax.fori_loop(0, C2, expand, 0)

            pltpu.async_copy(rows, acc.at[idx2.at[i]], sema, add=True)

        def drain(i, rows, sema):
            pltpu.make_async_copy(rows, acc.at[idx2.at[i]], sema).wait()

        def work(i, rowsA, exbA, semmA, semaA, rowsB, exbB, semmB, semaB):
            @pl.when(i >= 2)
            def _():
                drain(i - 2, rowsA, semaA)

            start(i, rowsA, exbA, semmA)

            @pl.when(i >= 1)
            def _():
                finish(i - 1, rowsB, exbB, semmB, semaB)

        def body(i, _):
            @pl.when(i % 2 == 0)
            def _():
                work(i, rows0, exb0, semm0, sema0, rows1, exb1, semm1, sema1)

            @pl.when(i % 2 == 1)
            def _():
                work(i, rows1, exb1, semm1, sema1, rows0, exb0, semm0, sema0)

            return 0

        lax.fori_loop(0, nch, body, 0)
        if (nch - 1) % 2 == 0:
            finish(nch - 1, rows0, exb0, semm0, sema0)
            drain(nch - 2, rows1, sema1)
            drain(nch - 1, rows0, sema0)
        else:
            finish(nch - 1, rows1, exb1, semm1, sema1)
            drain(nch - 2, rows0, sema0)
            drain(nch - 1, rows1, sema1)

    run_page(0)
    run_page(1)
    plsc.subcore_barrier()

    def dump(j, _):
        @pl.when(j % NS == s)
        def _():
            r0 = j * ZC
            pltpu.sync_copy(acc.at[pl.ds(r0, ZC)], rows0)
            pltpu.sync_copy(rows0, out_hbm.at[pl.ds(c * N_DST + r0, ZC)])
        return 0

    lax.fori_loop(0, NZCH, dump, 0)


def _sc_scatter(mn, ex_flat, dst2, zeros):
    mesh = plsc.VectorSubcoreMesh(core_axis_name="c", subcore_axis_name="s")
    f = functools.partial(
        pl.kernel,
        mesh=mesh,
        out_type=jax.ShapeDtypeStruct((NC * N_DST, D), jnp.float32),
        scratch_types=[
            pltpu.VMEM((EPS // 2 // C2, C2), jnp.int32),
            pltpu.VMEM((C2, D), jnp.float32),
            pltpu.VMEM((C2 * HEADS + 16,), jnp.float32),
            pltpu.VMEM((C2, D), jnp.float32),
            pltpu.VMEM((C2 * HEADS + 16,), jnp.float32),
            pltpu.SemaphoreType.DMA,
            pltpu.SemaphoreType.DMA,
            pltpu.SemaphoreType.DMA,
            pltpu.SemaphoreType.DMA,
            pltpu.VMEM_SHARED((N_DST, D), jnp.float32),
        ],
    )(_sc_scatter_body)
    return f(mn, ex_flat, dst2, zeros)


def _update_body(p1n_ref, p1d_ref, p2n_ref, p2d_ref, w0_ref, b0_ref,
                 w1_ref, b1_ref, rep_ref, out_ref):
    num = p1n_ref[...] + p2n_ref[...]
    den = p1d_ref[:, :HEADS] + p2d_ref[:, :HEADS]
    denw = jnp.dot(den, rep_ref[...],
                   preferred_element_type=jnp.float32) + 1e-16
    h = jnp.maximum(num / denw, 0.0)
    y0 = jnp.maximum(
        jnp.dot(h, w0_ref[...], preferred_element_type=jnp.float32)
        + b0_ref[...] + h, 0.0)
    out_ref[...] = jnp.maximum(
        jnp.dot(y0, w1_ref[...], preferred_element_type=jnp.float32)
        + b1_ref[...] + y0, 0.0)


def _tc_update(p1, p2, w0, b0, w1, b1, rep):
    blk = 1000
    grid = (N_DST // blk,)
    nb = N_DST // blk
    return pl.pallas_call(
        _update_body,
        grid=grid,
        in_specs=[
            pl.BlockSpec((blk, D), lambda i: (i, 0)),
            pl.BlockSpec((blk, D), lambda i: (i + nb, 0)),
            pl.BlockSpec((blk, D), lambda i: (i, 0)),
            pl.BlockSpec((blk, D), lambda i: (i + nb, 0)),
            pl.BlockSpec((D, D), lambda i: (0, 0)),
            pl.BlockSpec((1, D), lambda i: (0, 0)),
            pl.BlockSpec((D, D), lambda i: (0, 0)),
            pl.BlockSpec((1, D), lambda i: (0, 0)),
            pl.BlockSpec((HEADS, D), lambda i: (0, 0)),
        ],
        out_specs=pl.BlockSpec((blk, D), lambda i: (i, 0)),
        out_shape=jax.ShapeDtypeStruct((N_DST, D), jnp.float32),
    )(p1, p1, p2, p2, w0, b0, w1, b1, rep)


def kernel(x_src, x_dst, edge_attr, edge_index, q, kW0, kb0, kW1, kb1,
           vW0, vb0, vW1, vb1, oW0, ob0, oW1, ob1):
    f32 = jnp.float32
    wke = kW0[2 * D:]
    wve = vW0[2 * D:]
    scale = np.float32(1.0 / np.sqrt(float(D_HEAD)))
    qflat = q.reshape(D)
    sel = (jnp.arange(D)[:, None] // D_HEAD
           == jnp.arange(HEADS)[None, :]).astype(f32)
    qk = scale * ((kW1 + jnp.eye(D, dtype=f32)) @ (qflat[:, None] * sel))
    ck = (scale * jnp.sum((kb1 * qflat).reshape(HEADS, D_HEAD), axis=1)
          ).reshape(1, HEADS)
    rep = sel.T

    src = edge_index[0].astype(jnp.int32)
    dst = edge_index[1].astype(jnp.int32)

    ts32, td32 = _tc_tables(x_src, x_dst, kW0[:D], vW0[:D],
                            kW0[D:2 * D], vW0[D:2 * D],
                            kb0.reshape(1, D), vb0.reshape(1, D))
    zeros = jnp.zeros((ZC, D), f32)
    ps = []
    for h in range(1):
        sl = slice(h * EH, (h + 1) * EH)
        gs32, gd32 = _sc_gather(ts32, td32, src[sl], dst[sl])
        mn, ex8 = _tc_edges(gs32, gd32, edge_attr[sl], wke, wve, qk, ck, vW1,
                            vb1.reshape(1, D), rep)
        ps.append(_sc_scatter(mn, ex8.reshape(EH * HEADS),
                               dst[sl].reshape(NS * 2, EPS // 2 // C2, C2),
                               zeros))
    return _tc_update(ps[0], ps[0], oW0, ob0.reshape(1, D),
                      oW1, ob1.reshape(1, D), rep)


# speedup vs baseline: 1.2285x; 1.2285x over previous
import functools

import jax
import jax.numpy as jnp
import numpy as np
from jax import lax
from jax.experimental import pallas as pl
from jax.experimental.pallas import tpu as pltpu
from jax.experimental.pallas import tpu_sc as plsc

N_SRC = 10000
N_DST = 10000
E = 320000
D = 128
D_EDGE = 16
HEADS = 8
D_HEAD = 16
TW = 2 * D
TWW = TW // 2

NC, NS = 2, 16
NW = NC * NS
EH = E // 2
EPW = EH // NW
EPS = EH // NS

C1 = 80
T1 = EPW % C1
C2 = 80
ZC = 80
NZCH = N_DST // ZC


def _pack16(a, b):
    au = lax.bitcast_convert_type(a.astype(jnp.bfloat16), jnp.uint16)
    bu = lax.bitcast_convert_type(b.astype(jnp.bfloat16), jnp.uint16)
    word = au.astype(jnp.uint32) | (bu.astype(jnp.uint32) << 16)
    return lax.bitcast_convert_type(word, jnp.float32)


def _unpack16(w):
    word = lax.bitcast_convert_type(w, jnp.uint32)
    a = lax.bitcast_convert_type((word & 0xFFFF).astype(jnp.uint16),
                                 jnp.bfloat16)
    b = lax.bitcast_convert_type((word >> 16).astype(jnp.uint16),
                                 jnp.bfloat16)
    return a.astype(jnp.float32), b.astype(jnp.float32)


def _tables_body(xs_ref, xd_ref, wsk_ref, wsv_ref, wdk_ref, wdv_ref,
                 bk_ref, bv_ref, ts_ref, td_ref):
    xs = xs_ref[...]
    xd = xd_ref[...]
    ts_ref[...] = _pack16(
        jnp.dot(xs, wsk_ref[...], preferred_element_type=jnp.float32),
        jnp.dot(xs, wsv_ref[...], preferred_element_type=jnp.float32))
    td_ref[...] = _pack16(
        jnp.dot(xd, wdk_ref[...], preferred_element_type=jnp.float32)
        + bk_ref[...],
        jnp.dot(xd, wdv_ref[...], preferred_element_type=jnp.float32)
        + bv_ref[...])


def _tc_tables(x_src, x_dst, wsk, wsv, wdk, wdv, bk, bv):
    blk = 1000
    grid = (N_SRC // blk,)
    return pl.pallas_call(
        _tables_body,
        grid=grid,
        in_specs=[
            pl.BlockSpec((blk, D), lambda i: (i, 0)),
            pl.BlockSpec((blk, D), lambda i: (i, 0)),
            pl.BlockSpec((D, D), lambda i: (0, 0)),
            pl.BlockSpec((D, D), lambda i: (0, 0)),
            pl.BlockSpec((D, D), lambda i: (0, 0)),
            pl.BlockSpec((D, D), lambda i: (0, 0)),
            pl.BlockSpec((1, D), lambda i: (0, 0)),
            pl.BlockSpec((1, D), lambda i: (0, 0)),
        ],
        out_specs=[
            pl.BlockSpec((blk, TWW), lambda i: (i, 0)),
            pl.BlockSpec((blk, TWW), lambda i: (i, 0)),
        ],
        out_shape=[
            jax.ShapeDtypeStruct((N_SRC, TWW), jnp.float32),
            jax.ShapeDtypeStruct((N_DST, TWW), jnp.float32),
        ],
    )(x_src, x_dst, wsk, wsv, wdk, wdv, bk, bv)


def _sc_gather_body(ts_hbm, td_hbm, src_hbm, dst_hbm, gs_hbm, gd_hbm,
                    isa, ida, bs0, bd0, bs1, bd1,
                    semg0, semg1, semw0, semw1):
    wid = lax.axis_index("s") * NC + lax.axis_index("c")
    nch = EPW // C1
    base_w = wid * EPW
    pltpu.sync_copy(src_hbm.at[pl.ds(base_w, EPW)], isa)
    pltpu.sync_copy(dst_hbm.at[pl.ds(base_w, EPW)], ida)

    def start(i, bs, bd, semg):
        pltpu.async_copy(ts_hbm.at[isa.at[pl.ds(i * C1, C1)]], bs, semg)
        pltpu.async_copy(td_hbm.at[ida.at[pl.ds(i * C1, C1)]], bd, semg)

    def finish(i, bs, bd, semg, semw):
        pltpu.make_async_copy(
            ts_hbm.at[isa.at[pl.ds(i * C1, C1)]], bs, semg).wait()
        pltpu.make_async_copy(
            td_hbm.at[ida.at[pl.ds(i * C1, C1)]], bd, semg).wait()
        sl = pl.ds(base_w + i * C1, C1)
        pltpu.async_copy(bs, gs_hbm.at[sl], semw)
        pltpu.async_copy(bd, gd_hbm.at[sl], semw)

    def drain(i, bs, bd, semw):
        sl = pl.ds(base_w + i * C1, C1)
        pltpu.make_async_copy(bs, gs_hbm.at[sl], semw).wait()
        pltpu.make_async_copy(bd, gd_hbm.at[sl], semw).wait()

    def work(i, bsA, bdA, semgA, semwA, bsB, bdB, semgB, semwB):
        @pl.when(i >= 2)
        def _():
            drain(i - 2, bsA, bdA, semwA)

        start(i, bsA, bdA, semgA)

        @pl.when(i >= 1)
        def _():
            finish(i - 1, bsB, bdB, semgB, semwB)

    def body(i, _):
        @pl.when(i % 2 == 0)
        def _():
            work(i, bs0, bd0, semg0, semw0, bs1, bd1, semg1, semw1)

        @pl.when(i % 2 == 1)
        def _():
            work(i, bs1, bd1, semg1, semw1, bs0, bd0, semg0, semw0)

        return 0

    lax.fori_loop(0, nch, body, 0)
    if (nch - 1) % 2 == 0:
        finish(nch - 1, bs0, bd0, semg0, semw0)
        drain(nch - 2, bs1, bd1, semw1)
        drain(nch - 1, bs0, bd0, semw0)
    else:
        finish(nch - 1, bs1, bd1, semg1, semw1)
        drain(nch - 2, bs0, bd0, semw0)
        drain(nch - 1, bs1, bd1, semw1)
    if T1:
        t0 = nch * C1
        pltpu.async_copy(ts_hbm.at[isa.at[pl.ds(t0, T1)]],
                         bs0.at[pl.ds(0, T1)], semg0).wait()
        pltpu.async_copy(td_hbm.at[ida.at[pl.ds(t0, T1)]],
                         bd0.at[pl.ds(0, T1)], semg0).wait()
        pltpu.sync_copy(bs0.at[pl.ds(0, T1)], gs_hbm.at[pl.ds(base_w + t0, T1)])
        pltpu.sync_copy(bd0.at[pl.ds(0, T1)], gd_hbm.at[pl.ds(base_w + t0, T1)])


def _sc_gather(ts, td, src, dst):
    mesh = plsc.VectorSubcoreMesh(core_axis_name="c", subcore_axis_name="s")
    f = functools.partial(
        pl.kernel,
        mesh=mesh,
        out_type=[
            jax.ShapeDtypeStruct((EH, TWW), jnp.float32),
            jax.ShapeDtypeStruct((EH, TWW), jnp.float32),
        ],
        scratch_types=[
            pltpu.VMEM((EPW,), jnp.int32),
            pltpu.VMEM((EPW,), jnp.int32),
            pltpu.VMEM((C1, TWW), jnp.float32),
            pltpu.VMEM((C1, TWW), jnp.float32),
            pltpu.VMEM((C1, TWW), jnp.float32),
            pltpu.VMEM((C1, TWW), jnp.float32),
            pltpu.SemaphoreType.DMA,
            pltpu.SemaphoreType.DMA,
            pltpu.SemaphoreType.DMA,
            pltpu.SemaphoreType.DMA,
        ],
    )(_sc_gather_body)
    return f(ts, td, src, dst)


def _edge_body(gs_ref, gd_ref, ea_ref, wke_ref, wve_ref, qk_ref, ck_ref,
               wv1_ref, bv1_ref, rep_ref, mn_ref, ex_ref):
    ea = ea_ref[...]
    ks, vs = _unpack16(gs_ref[...])
    kd, vd = _unpack16(gd_ref[...])
    h0k = jnp.maximum(
        ks + kd
        + jnp.dot(ea, wke_ref[...], preferred_element_type=jnp.float32), 0.0)
    coef = jnp.dot(h0k, qk_ref[...],
                   preferred_element_type=jnp.float32) + ck_ref[...]
    ex = jnp.exp(coef)
    h0v = jnp.maximum(
        vs + vd
        + jnp.dot(ea, wve_ref[...], preferred_element_type=jnp.float32), 0.0)
    v1 = jnp.dot(h0v, wv1_ref[...],
                 preferred_element_type=jnp.float32) + bv1_ref[...] + h0v
    exw = jnp.dot(ex, rep_ref[...],
                  preferred_element_type=jnp.float32)
    mn_ref[...] = exw * v1
    ex_ref[...] = ex


def _tc_edges(gs, gd, ea, wke, wve, qk, ck, wv1, bv1, rep):
    blk = 2000
    grid = (EH // blk,)
    return pl.pallas_call(
        _edge_body,
        grid=grid,
        in_specs=[
            pl.BlockSpec((blk, TWW), lambda i: (i, 0)),
            pl.BlockSpec((blk, TWW), lambda i: (i, 0)),
            pl.BlockSpec((blk, D_EDGE), lambda i: (i, 0)),
            pl.BlockSpec((D_EDGE, D), lambda i: (0, 0)),
            pl.BlockSpec((D_EDGE, D), lambda i: (0, 0)),
            pl.BlockSpec((D, HEADS), lambda i: (0, 0)),
            pl.BlockSpec((1, HEADS), lambda i: (0, 0)),
            pl.BlockSpec((D, D), lambda i: (0, 0)),
            pl.BlockSpec((1, D), lambda i: (0, 0)),
            pl.BlockSpec((HEADS, D), lambda i: (0, 0)),
        ],
        out_specs=[
            pl.BlockSpec((blk, D), lambda i: (i, 0)),
            pl.BlockSpec((blk, HEADS), lambda i: (i, 0)),
        ],
        out_shape=[
            jax.ShapeDtypeStruct((EH, D), jnp.float32),
            jax.ShapeDtypeStruct((EH, HEADS), jnp.float32),
        ],
    )(gs, gd, ea, wke, wve, qk, ck, wv1, bv1, rep)


def _sc_scatter_body(mn_hbm, ex_hbm, dst2_hbm, z_hbm, out_hbm,
                     idx2, rows0, exb0, rows1, exb1,
                     semm0, semm1, sema0, sema1, acc):
    c = lax.axis_index("c")
    s = lax.axis_index("s")
    nch = EPS // C2

    pltpu.sync_copy(dst2_hbm.at[s], idx2)
    pltpu.sync_copy(z_hbm, rows0)
    pltpu.sync_copy(z_hbm, rows1)

    def zinit(j, _):
        @pl.when(j % NS == s)
        def _():
            pltpu.sync_copy(rows0, acc.at[pl.ds(j * ZC, ZC)])
        return 0

    lax.fori_loop(0, NZCH, zinit, 0)
    plsc.subcore_barrier()

    low = lax.iota(jnp.int32, 16) < HEADS

    def start(i, rows, exb, semm):
        base = s * EPS + i * C2

        @pl.when(c == 0)
        def _():
            pltpu.async_copy(mn_hbm.at[pl.ds(base, C2)], rows, semm)

        @pl.when(c == 1)
        def _():
            pltpu.async_copy(ex_hbm.at[pl.ds(base * HEADS, C2 * HEADS)],
                             exb.at[pl.ds(0, C2 * HEADS)], semm)

    def finish(i, rows, exb, semm, sema):
        base = s * EPS + i * C2

        @pl.when(c == 0)
        def _():
            pltpu.make_async_copy(
                mn_hbm.at[pl.ds(base, C2)], rows, semm).wait()

        @pl.when(c == 1)
        def _():
            pltpu.make_async_copy(
                ex_hbm.at[pl.ds(base * HEADS, C2 * HEADS)],
                exb.at[pl.ds(0, C2 * HEADS)], semm).wait()

            def expand(r, _):
                vec = jnp.where(low, exb[pl.ds(r * HEADS, 16)], 0.0)
                rows[r, pl.ds(0, 16)] = vec
                return 0

            lax.fori_loop(0, C2, expand, 0)

        pltpu.async_copy(rows, acc.at[idx2.at[i]], sema, add=True)

    def drain(i, rows, sema):
        pltpu.make_async_copy(rows, acc.at[idx2.at[i]], sema).wait()

    def work(i, rowsA, exbA, semmA, semaA, rowsB, exbB, semmB, semaB):
        @pl.when(i >= 2)
        def _():
            drain(i - 2, rowsA, semaA)

        start(i, rowsA, exbA, semmA)

        @pl.when(i >= 1)
        def _():
            finish(i - 1, rowsB, exbB, semmB, semaB)

    def body(i, _):
        @pl.when(i % 2 == 0)
        def _():
            work(i, rows0, exb0, semm0, sema0, rows1, exb1, semm1, sema1)

        @pl.when(i % 2 == 1)
        def _():
            work(i, rows1, exb1, semm1, sema1, rows0, exb0, semm0, sema0)

        return 0

    lax.fori_loop(0, nch, body, 0)
    if (nch - 1) % 2 == 0:
        finish(nch - 1, rows0, exb0, semm0, sema0)
        drain(nch - 2, rows1, sema1)
        drain(nch - 1, rows0, sema0)
    else:
        finish(nch - 1, rows1, exb1, semm1, sema1)
        drain(nch - 2, rows0, sema0)
        drain(nch - 1, rows1, sema1)
    plsc.subcore_barrier()

    def dump(j, _):
        @pl.when(j % NS == s)
        def _():
            r0 = j * ZC
            pltpu.sync_copy(acc.at[pl.ds(r0, ZC)], rows0)
            pltpu.sync_copy(rows0, out_hbm.at[pl.ds(c * N_DST + r0, ZC)])
        return 0

    lax.fori_loop(0, NZCH, dump, 0)


def _sc_scatter(mn, ex_flat, dst2, zeros):
    mesh = plsc.VectorSubcoreMesh(core_axis_name="c", subcore_axis_name="s")
    f = functools.partial(
        pl.kernel,
        mesh=mesh,
        out_type=jax.ShapeDtypeStruct((NC * N_DST, D), jnp.float32),
        scratch_types=[
            pltpu.VMEM((EPS // C2, C2), jnp.int32),
            pltpu.VMEM((C2, D), jnp.float32),
            pltpu.VMEM((C2 * HEADS + 16,), jnp.float32),
            pltpu.VMEM((C2, D), jnp.float32),
            pltpu.VMEM((C2 * HEADS + 16,), jnp.float32),
            pltpu.SemaphoreType.DMA,
            pltpu.SemaphoreType.DMA,
            pltpu.SemaphoreType.DMA,
            pltpu.SemaphoreType.DMA,
            pltpu.VMEM_SHARED((N_DST, D), jnp.float32),
        ],
    )(_sc_scatter_body)
    return f(mn, ex_flat, dst2, zeros)


def _update_body(p1n_ref, p1d_ref, p2n_ref, p2d_ref, w0_ref, b0_ref,
                 w1_ref, b1_ref, rep_ref, out_ref):
    num = p1n_ref[...] + p2n_ref[...]
    den = p1d_ref[:, :HEADS] + p2d_ref[:, :HEADS]
    denw = jnp.dot(den, rep_ref[...],
                   preferred_element_type=jnp.float32) + 1e-16
    h = jnp.maximum(num / denw, 0.0)
    y0 = jnp.maximum(
        jnp.dot(h, w0_ref[...], preferred_element_type=jnp.float32)
        + b0_ref[...] + h, 0.0)
    out_ref[...] = jnp.maximum(
        jnp.dot(y0, w1_ref[...], preferred_element_type=jnp.float32)
        + b1_ref[...] + y0, 0.0)


def _tc_update(p1, p2, w0, b0, w1, b1, rep):
    blk = 1000
    grid = (N_DST // blk,)
    nb = N_DST // blk
    return pl.pallas_call(
        _update_body,
        grid=grid,
        in_specs=[
            pl.BlockSpec((blk, D), lambda i: (i, 0)),
            pl.BlockSpec((blk, D), lambda i: (i + nb, 0)),
            pl.BlockSpec((blk, D), lambda i: (i, 0)),
            pl.BlockSpec((blk, D), lambda i: (i + nb, 0)),
            pl.BlockSpec((D, D), lambda i: (0, 0)),
            pl.BlockSpec((1, D), lambda i: (0, 0)),
            pl.BlockSpec((D, D), lambda i: (0, 0)),
            pl.BlockSpec((1, D), lambda i: (0, 0)),
            pl.BlockSpec((HEADS, D), lambda i: (0, 0)),
        ],
        out_specs=pl.BlockSpec((blk, D), lambda i: (i, 0)),
        out_shape=jax.ShapeDtypeStruct((N_DST, D), jnp.float32),
    )(p1, p1, p2, p2, w0, b0, w1, b1, rep)


def kernel(x_src, x_dst, edge_attr, edge_index, q, kW0, kb0, kW1, kb1,
           vW0, vb0, vW1, vb1, oW0, ob0, oW1, ob1):
    f32 = jnp.float32
    wke = kW0[2 * D:]
    wve = vW0[2 * D:]
    scale = np.float32(1.0 / np.sqrt(float(D_HEAD)))
    qflat = q.reshape(D)
    sel = (jnp.arange(D)[:, None] // D_HEAD
           == jnp.arange(HEADS)[None, :]).astype(f32)
    qk = scale * ((kW1 + jnp.eye(D, dtype=f32)) @ (qflat[:, None] * sel))
    ck = (scale * jnp.sum((kb1 * qflat).reshape(HEADS, D_HEAD), axis=1)
          ).reshape(1, HEADS)
    rep = sel.T

    src = edge_index[0].astype(jnp.int32)
    dst = edge_index[1].astype(jnp.int32)

    ts32, td32 = _tc_tables(x_src, x_dst, kW0[:D], vW0[:D],
                            kW0[D:2 * D], vW0[D:2 * D],
                            kb0.reshape(1, D), vb0.reshape(1, D))
    zeros = jnp.zeros((ZC, D), f32)
    ps = []
    for h in range(2):
        sl = slice(h * EH, (h + 1) * EH)
        gs32, gd32 = _sc_gather(ts32, td32, src[sl], dst[sl])
        mn, ex8 = _tc_edges(gs32, gd32, edge_attr[sl], wke, wve, qk, ck, vW1,
                            vb1.reshape(1, D), rep)
        ps.append(_sc_scatter(mn, ex8.reshape(EH * HEADS),
                               dst[sl].reshape(NS, EPS // C2, C2),
                               zeros))
    return _tc_update(ps[0], ps[1], oW0, ob0.reshape(1, D),
                      oW1, ob1.reshape(1, D), rep)
